# baseline (speedup 1.0000x reference)
import functools

import jax
import jax.numpy as jnp
from jax import lax
from jax.experimental import pallas as pl
from jax.experimental.pallas import tpu as pltpu
from jax.experimental.pallas import tpu_sc as plsc

N = 10000
E = 320000
C = 128
K = 27

CH = 128
NCHUNK = E // CH
NW = 32
ITERS = -(-NCHUNK // NW)
NPAD = 10112
RT = NPAD // 16
LANES = 16


def _sc_conv_body(h_hbm, idx_hbm, out_hbm,
                  idx_v, flat_v, rows_v, acc_sh, sem):
    c = lax.axis_index("c")
    s = lax.axis_index("s")
    wid = c * 16 + s

    def _zero_rows(i, carry):
        for j in range(C // LANES):
            rows_v[i, pl.ds(j * LANES, LANES)] = jnp.zeros((LANES,), jnp.float32)
        return carry

    lax.fori_loop(0, CH, _zero_rows, 0)

    r0 = s * RT
    for t in range(RT // CH):
        pltpu.sync_copy(rows_v, acc_sh.at[pl.ds(r0 + t * CH, CH)])
    rem = RT % CH
    if rem:
        pltpu.sync_copy(rows_v.at[pl.ds(0, rem)],
                        acc_sh.at[pl.ds(r0 + (RT // CH) * CH, rem)])

    plsc.subcore_barrier()

    def _body(i, carry):
        chunk = i * NW + wid

        @pl.when(chunk < NCHUNK)
        def _():
            pltpu.sync_copy(idx_hbm.at[chunk], idx_v)
            for j in range(CH // LANES):
                sl = pl.ds(j * LANES, LANES)
                flat_v[sl] = idx_v[1, sl] * N + idx_v[0, sl]
            pltpu.async_copy(h_hbm.at[flat_v], rows_v, sem).wait()
            pltpu.sync_copy(rows_v, acc_sh.at[idx_v.at[2]], add=True)

        return carry

    lax.fori_loop(0, ITERS, _body, 0)

    plsc.subcore_barrier()

    pltpu.sync_copy(acc_sh.at[pl.ds(r0, RT)],
                    out_hbm.at[pl.ds(c * NPAD + r0, RT)])


_sc_conv = functools.partial(
    pl.kernel,
    out_type=jax.ShapeDtypeStruct((2 * NPAD, C), jnp.float32),
    mesh=plsc.VectorSubcoreMesh(core_axis_name="c", subcore_axis_name="s"),
    scratch_types=[
        pltpu.VMEM((3, CH), jnp.int32),
        pltpu.VMEM((CH,), jnp.int32),
        pltpu.VMEM((CH, C), jnp.float32),
        pltpu.VMEM_SHARED((NPAD, C), jnp.float32),
        pltpu.SemaphoreType.DMA,
    ],
)(_sc_conv_body)


def _mm_kernel(x_ref, w_ref, o_ref):
    o_ref[0] = jnp.dot(x_ref[...], w_ref[0],
                       preferred_element_type=jnp.float32)


_MM_NB = 10
_MM_BN = N // _MM_NB


def _matmul(x, w):
    return pl.pallas_call(
        _mm_kernel,
        grid=(_MM_NB, K),
        in_specs=[
            pl.BlockSpec((_MM_BN, C), lambda j, k: (j, 0)),
            pl.BlockSpec((1, C, C), lambda j, k: (k, 0, 0)),
        ],
        out_specs=pl.BlockSpec((1, _MM_BN, C), lambda j, k: (k, j, 0)),
        out_shape=jax.ShapeDtypeStruct((K, N, C), jnp.float32),
    )(x, w)


_ST_NB = 8
_ST_BR = NPAD // _ST_NB
_NM_NB = 10
_NM_BR = N // _NM_NB


def _stats_kernel(p_ref, o_ref):
    s = p_ref[0] + p_ref[1]
    ps = jnp.sum(s, axis=0, keepdims=True)
    pq = jnp.sum(s * s, axis=0, keepdims=True)

    @pl.when(pl.program_id(0) == 0)
    def _():
        o_ref[...] = jnp.zeros_like(o_ref)

    o_ref[0:1, :] += ps
    o_ref[1:2, :] += pq


def _stats(p):
    return pl.pallas_call(
        _stats_kernel,
        grid=(_ST_NB,),
        in_specs=[pl.BlockSpec((2, _ST_BR, C), lambda j: (0, j, 0))],
        out_specs=pl.BlockSpec((8, C), lambda j: (0, 0)),
        out_shape=jax.ShapeDtypeStruct((8, C), jnp.float32),
    )(p)


def _affine(st_ref, g_ref, b_ref):
    mu = st_ref[0:1, :] * (1.0 / N)
    var = st_ref[1:2, :] * (1.0 / N) - mu * mu
    scale = g_ref[...] * lax.rsqrt(var + 1e-5)
    shift = b_ref[...] - mu * scale
    return scale, shift


def _norm_relu_kernel(p_ref, st_ref, g_ref, b_ref, o_ref):
    scale, shift = _affine(st_ref, g_ref, b_ref)
    s = p_ref[0] + p_ref[1]
    o_ref[...] = jnp.maximum(s * scale + shift, 0.0)


def _norm_res_relu_kernel(p_ref, st_ref, x_ref, g_ref, b_ref, o_ref):
    scale, shift = _affine(st_ref, g_ref, b_ref)
    s = p_ref[0] + p_ref[1]
    o_ref[...] = jnp.maximum(s * scale + shift + x_ref[...], 0.0)


def _bn_relu(p, gamma, beta):
    st = _stats(p)
    return pl.pallas_call(
        _norm_relu_kernel,
        grid=(_NM_NB,),
        in_specs=[
            pl.BlockSpec((2, _NM_BR, C), lambda j: (0, j, 0)),
            pl.BlockSpec((8, C), lambda j: (0, 0)),
            pl.BlockSpec((1, C), lambda j: (0, 0)),
            pl.BlockSpec((1, C), lambda j: (0, 0)),
        ],
        out_specs=pl.BlockSpec((_NM_BR, C), lambda j: (j, 0)),
        out_shape=jax.ShapeDtypeStruct((N, C), jnp.float32),
    )(p, st, gamma.reshape(1, C), beta.reshape(1, C))


def _bn_res_relu(p, x, gamma, beta):
    st = _stats(p)
    return pl.pallas_call(
        _norm_res_relu_kernel,
        grid=(_NM_NB,),
        in_specs=[
            pl.BlockSpec((2, _NM_BR, C), lambda j: (0, j, 0)),
            pl.BlockSpec((8, C), lambda j: (0, 0)),
            pl.BlockSpec((_NM_BR, C), lambda j: (j, 0)),
            pl.BlockSpec((1, C), lambda j: (0, 0)),
            pl.BlockSpec((1, C), lambda j: (0, 0)),
        ],
        out_specs=pl.BlockSpec((_NM_BR, C), lambda j: (j, 0)),
        out_shape=jax.ShapeDtypeStruct((N, C), jnp.float32),
    )(p, st, x, gamma.reshape(1, C), beta.reshape(1, C))


def kernel(x, edge_index, kernel_idx, W1, gamma1, beta1, W2, gamma2, beta2):
    idx = jnp.stack(
        [edge_index[0].reshape(NCHUNK, CH),
         kernel_idx.reshape(NCHUNK, CH),
         edge_index[1].reshape(NCHUNK, CH)], axis=1)

    h1 = _matmul(x, W1).reshape(K * N, C)
    p1 = _sc_conv(h1, idx).reshape(2, NPAD, C)
    out1 = _bn_relu(p1, gamma1, beta1)

    h2 = _matmul(out1, W2).reshape(K * N, C)
    p2 = _sc_conv(h2, idx).reshape(2, NPAD, C)
    out = _bn_res_relu(p2, x, gamma2, beta2)
    return out


# speedup vs baseline: 1.0180x; 1.0180x over previous
import functools

import jax
import jax.numpy as jnp
from jax import lax
from jax.experimental import pallas as pl
from jax.experimental.pallas import tpu as pltpu
from jax.experimental.pallas import tpu_sc as plsc

N = 10000
E = 320000
C = 128
K = 27

CH = 128
NCHUNK = E // CH
NW = 32
ITERS = -(-NCHUNK // NW)
NPAD = 10112
RT = NPAD // 16
LANES = 16


def _sc_conv_body(h_hbm, idx_hbm, out_hbm,
                  idx_v, flat_v, rows_v, acc_sh, sem):
    c = lax.axis_index("c")
    s = lax.axis_index("s")
    wid = c * 16 + s

    def _zero_rows(i, carry):
        for j in range(C // LANES):
            rows_v[i, pl.ds(j * LANES, LANES)] = jnp.zeros((LANES,), jnp.float32)
        return carry

    lax.fori_loop(0, CH, _zero_rows, 0)

    r0 = s * RT
    for t in range(RT // CH):
        pltpu.sync_copy(rows_v, acc_sh.at[pl.ds(r0 + t * CH, CH)])
    rem = RT % CH
    if rem:
        pltpu.sync_copy(rows_v.at[pl.ds(0, rem)],
                        acc_sh.at[pl.ds(r0 + (RT // CH) * CH, rem)])

    plsc.subcore_barrier()

    def _body(i, carry):
        chunk = i * NW + wid

        @pl.when(chunk < NCHUNK)
        def _():
            pltpu.sync_copy(idx_hbm.at[chunk], idx_v)
            for j in range(CH // LANES):
                sl = pl.ds(j * LANES, LANES)
                flat_v[sl] = idx_v[1, sl] * N + idx_v[0, sl]
            pltpu.async_copy(h_hbm.at[flat_v], rows_v, sem).wait()
            pltpu.sync_copy(rows_v, acc_sh.at[idx_v.at[2]], add=True)

        return carry

    lax.fori_loop(0, ITERS, _body, 0)

    plsc.subcore_barrier()

    pltpu.sync_copy(acc_sh.at[pl.ds(r0, RT)],
                    out_hbm.at[pl.ds(c * NPAD + r0, RT)])


_sc_conv = functools.partial(
    pl.kernel,
    out_type=jax.ShapeDtypeStruct((2 * NPAD, C), jnp.float32),
    mesh=plsc.VectorSubcoreMesh(core_axis_name="c", subcore_axis_name="s"),
    scratch_types=[
        pltpu.VMEM((3, CH), jnp.int32),
        pltpu.VMEM((CH,), jnp.int32),
        pltpu.VMEM((CH, C), jnp.float32),
        pltpu.VMEM_SHARED((NPAD, C), jnp.float32),
        pltpu.SemaphoreType.DMA,
    ],
)(_sc_conv_body)


def _mm_kernel(x_ref, w_ref, o_ref):
    o_ref[0] = jnp.dot(x_ref[...], w_ref[0],
                       preferred_element_type=jnp.float32)


_MM_NB = 10
_MM_BN = N // _MM_NB


def _matmul(x, w):
    return pl.pallas_call(
        _mm_kernel,
        grid=(_MM_NB, K),
        in_specs=[
            pl.BlockSpec((_MM_BN, C), lambda j, k: (j, 0)),
            pl.BlockSpec((1, C, C), lambda j, k: (k, 0, 0)),
        ],
        out_specs=pl.BlockSpec((1, _MM_BN, C), lambda j, k: (k, j, 0)),
        out_shape=jax.ShapeDtypeStruct((K, N, C), jnp.float32),
    )(x, w)


def _bn_relu_kernel(p_ref, g_ref, b_ref, o_ref):
    s = p_ref[0, :N, :] + p_ref[1, :N, :]
    mu = jnp.mean(s, axis=0, keepdims=True)
    var = jnp.mean(jnp.square(s - mu), axis=0, keepdims=True)
    y = g_ref[...] * (s - mu) * lax.rsqrt(var + 1e-5) + b_ref[...]
    o_ref[...] = jnp.maximum(y, 0.0)


def _bn_res_relu_kernel(p_ref, x_ref, g_ref, b_ref, o_ref):
    s = p_ref[0, :N, :] + p_ref[1, :N, :]
    mu = jnp.mean(s, axis=0, keepdims=True)
    var = jnp.mean(jnp.square(s - mu), axis=0, keepdims=True)
    y = g_ref[...] * (s - mu) * lax.rsqrt(var + 1e-5) + b_ref[...]
    o_ref[...] = jnp.maximum(y + x_ref[...], 0.0)


def _bn_relu(p, gamma, beta):
    return pl.pallas_call(
        _bn_relu_kernel,
        out_shape=jax.ShapeDtypeStruct((N, C), jnp.float32),
    )(p, gamma.reshape(1, C), beta.reshape(1, C))


def _bn_res_relu(p, x, gamma, beta):
    return pl.pallas_call(
        _bn_res_relu_kernel,
        out_shape=jax.ShapeDtypeStruct((N, C), jnp.float32),
    )(p, x, gamma.reshape(1, C), beta.reshape(1, C))


def kernel(x, edge_index, kernel_idx, W1, gamma1, beta1, W2, gamma2, beta2):
    idx = jnp.stack(
        [edge_index[0].reshape(NCHUNK, CH),
         kernel_idx.reshape(NCHUNK, CH),
         edge_index[1].reshape(NCHUNK, CH)], axis=1)

    h1 = _matmul(x, W1).reshape(K * N, C)
    p1 = _sc_conv(h1, idx).reshape(2, NPAD, C)
    out1 = _bn_relu(p1, gamma1, beta1)

    h2 = _matmul(out1, W2).reshape(K * N, C)
    p2 = _sc_conv(h2, idx).reshape(2, NPAD, C)
    out = _bn_res_relu(p2, x, gamma2, beta2)
    return out
